# Initial kernel scaffold; baseline (speedup 1.0000x reference)
import functools

import jax
import jax.numpy as jnp
from jax import lax
from jax.experimental import pallas as pl
from jax.experimental.pallas import tpu as pltpu
from jax.experimental.pallas import tpu_sc as plsc

F32 = jnp.float32
_NC = 2
_NS = 16
_CK = 80


def _mesh():
    return plsc.VectorSubcoreMesh(core_axis_name="c", subcore_axis_name="s")


@functools.cache
def _sc_degree(E, NP):
    E2 = E // _NC
    EP = E2 // _NS
    NCH = EP // _CK
    RP = NP // _NS

    @functools.partial(
        pl.kernel,
        out_type=jax.ShapeDtypeStruct((_NC, NP, 16), F32),
        mesh=_mesh(),
        scratch_types=[
            pltpu.VMEM_SHARED((NP, 16), F32),
            pltpu.VMEM((_CK,), jnp.int32),
            pltpu.VMEM((_CK, 16), F32),
            pltpu.VMEM((RP, 16), F32),
        ],
    )
    def k(dst_hbm, out_hbm, acc, idx_v, ones_v, zero_v):
        c = lax.axis_index("c")
        s = lax.axis_index("s")

        @pl.loop(0, _CK)
        def _(i):
            ones_v[i, :] = jnp.ones((16,), F32)

        @pl.loop(0, RP)
        def _(i):
            zero_v[i, :] = jnp.zeros((16,), F32)

        pltpu.sync_copy(zero_v, acc.at[pl.ds(s * RP, RP)])
        plsc.subcore_barrier()

        base0 = c * E2 + s * EP

        @pl.loop(0, NCH)
        def _(j):
            pltpu.sync_copy(dst_hbm.at[pl.ds(base0 + j * _CK, _CK)], idx_v)
            pltpu.sync_copy(ones_v, acc.at[idx_v], add=True)

        plsc.subcore_barrier()
        pltpu.sync_copy(acc.at[pl.ds(s * RP, RP)],
                        out_hbm.at[c, pl.ds(s * RP, RP)])

    return k


@functools.cache
def _sc_scatter(N, D, E):
    E2 = E // _NC
    EP = E2 // _NS
    NCH = EP // _CK
    RP = N // _NS
    ZR = 125

    @functools.partial(
        pl.kernel,
        out_type=jax.ShapeDtypeStruct((_NC, N, D), F32),
        mesh=_mesh(),
        scratch_types=[
            pltpu.VMEM_SHARED((N, D), F32),
            pltpu.VMEM((_CK,), jnp.int32),
            pltpu.VMEM((_CK,), jnp.int32),
            pltpu.VMEM((_CK, D), F32),
            pltpu.VMEM((ZR, D), F32),
            pltpu.SemaphoreType.DMA,
        ],
    )
    def k(t_hbm, src_hbm, dst_hbm, out_hbm, acc, src_v, dst_v, rows_v, zero_v, sem):
        c = lax.axis_index("c")
        s = lax.axis_index("s")

        @pl.loop(0, ZR)
        def _(i):
            @pl.loop(0, D // 16)
            def _(j):
                zero_v[i, pl.ds(j * 16, 16)] = jnp.zeros((16,), F32)

        for kk in range(RP // ZR):
            pltpu.sync_copy(zero_v, acc.at[pl.ds(s * RP + kk * ZR, ZR)])
        plsc.subcore_barrier()

        base0 = c * E2 + s * EP

        @pl.loop(0, NCH)
        def _(j):
            b = base0 + j * _CK
            pltpu.sync_copy(src_hbm.at[pl.ds(b, _CK)], src_v)
            pltpu.sync_copy(dst_hbm.at[pl.ds(b, _CK)], dst_v)
            pltpu.async_copy(t_hbm.at[src_v], rows_v, sem).wait()
            pltpu.sync_copy(rows_v, acc.at[dst_v], add=True)

        plsc.subcore_barrier()
        for kk in range(RP // ZR):
            r0 = s * RP + kk * ZR
            pltpu.sync_copy(acc.at[pl.ds(r0, ZR)], out_hbm.at[c, pl.ds(r0, ZR)])

    return k


def _tc_dis(degp, N, D):
    def body(degp_ref, o_ref):
        d = degp_ref[0, :, 0:1] + degp_ref[1, :, 0:1] + 1.0
        o_ref[...] = jnp.broadcast_to(lax.rsqrt(d)[:N], (N, D))

    return pl.pallas_call(
        body, out_shape=jax.ShapeDtypeStruct((N, D), F32))(degp)


def _tc_layer0(x, dis_b, W):
    def body(x_ref, dis_ref, W_ref, o_ref):
        o_ref[...] = jnp.dot(x_ref[...], W_ref[...],
                             preferred_element_type=F32) * dis_ref[...]

    return pl.pallas_call(
        body, out_shape=jax.ShapeDtypeStruct(x.shape, F32))(x, dis_b, W)


def _tc_layer(S, t, dis_b, b, W):
    def body(S_ref, t_ref, dis_ref, b_ref, W_ref, o_ref):
        agg = (S_ref[0] + S_ref[1] + t_ref[...]) * dis_ref[...]
        h = jnp.maximum(agg + b_ref[...], 0.0)
        o_ref[...] = jnp.dot(h, W_ref[...],
                             preferred_element_type=F32) * dis_ref[...]

    return pl.pallas_call(
        body, out_shape=jax.ShapeDtypeStruct(t.shape, F32))(S, t, dis_b, b, W)


def _tc_head(S, t, dis_b, b, head_W, head_b):
    N = t.shape[0]

    def body(S_ref, t_ref, dis_ref, b_ref, hW_ref, hb_ref, o_ref):
        agg = (S_ref[0] + S_ref[1] + t_ref[...]) * dis_ref[...]
        h = jnp.maximum(agg + b_ref[...], 0.0)
        o_ref[...] = jnp.maximum(
            jnp.dot(h, hW_ref[...], preferred_element_type=F32) + hb_ref[...],
            0.0)

    return pl.pallas_call(
        body, out_shape=jax.ShapeDtypeStruct((N, 1), F32))(
            S, t, dis_b, b, head_W, head_b)


def kernel(x, edge_index, Ws, bs, head_W, head_b):
    N, D = x.shape
    E = edge_index.shape[1]
    L = Ws.shape[0]
    src = edge_index[0]
    dst = edge_index[1]

    NP = -(-N // (16 * _NS)) * (16 * _NS)

    degp = _sc_degree(E, NP)(dst)
    dis_b = _tc_dis(degp, N, D)

    t = _tc_layer0(x, dis_b, Ws[0])
    scat = _sc_scatter(N, D, E)
    out = None
    for i in range(L):
        S = scat(t, src, dst)
        b = bs[i].reshape(1, D)
        if i + 1 < L:
            t = _tc_layer(S, t, dis_b, b, Ws[i + 1])
        else:
            out = _tc_head(S, t, dis_b, b, head_W, head_b.reshape(1, 1))
    return out


# speedup vs baseline: 7.8966x; 7.8966x over previous
import functools

import jax
import jax.numpy as jnp
from jax import lax
from jax.experimental import pallas as pl
from jax.experimental.pallas import tpu as pltpu
from jax.experimental.pallas import tpu_sc as plsc

F32 = jnp.float32
_NC = 2
_NS = 16
_CK = 80


def _mesh():
    return plsc.VectorSubcoreMesh(core_axis_name="c", subcore_axis_name="s")


@functools.cache
def _sc_degree(E, NP):
    E2 = E // _NC
    EP = E2 // _NS
    NCH = EP // _CK
    RP = NP // _NS

    @functools.partial(
        pl.kernel,
        out_type=jax.ShapeDtypeStruct((_NC, NP, 16), F32),
        mesh=_mesh(),
        scratch_types=[
            pltpu.VMEM_SHARED((NP, 16), F32),
            pltpu.VMEM((_CK,), jnp.int32),
            pltpu.VMEM((_CK, 16), F32),
            pltpu.VMEM((RP, 16), F32),
        ],
    )
    def k(dst_hbm, out_hbm, acc, idx_v, ones_v, zero_v):
        c = lax.axis_index("c")
        s = lax.axis_index("s")

        @pl.loop(0, _CK)
        def _(i):
            ones_v[i, :] = jnp.ones((16,), F32)

        @pl.loop(0, RP)
        def _(i):
            zero_v[i, :] = jnp.zeros((16,), F32)

        pltpu.sync_copy(zero_v, acc.at[pl.ds(s * RP, RP)])
        plsc.subcore_barrier()

        base0 = c * E2 + s * EP

        @pl.loop(0, NCH)
        def _(j):
            pltpu.sync_copy(dst_hbm.at[pl.ds(base0 + j * _CK, _CK)], idx_v)
            pltpu.sync_copy(ones_v, acc.at[idx_v], add=True)

        plsc.subcore_barrier()
        pltpu.sync_copy(acc.at[pl.ds(s * RP, RP)],
                        out_hbm.at[c, pl.ds(s * RP, RP)])

    return k


@functools.cache
def _sc_scatter(NP, D, E):
    E2 = E // _NC
    EP = E2 // _NS
    NCH = EP // _CK
    RP = NP // _NS
    ZR = 128

    @functools.partial(
        pl.kernel,
        out_type=jax.ShapeDtypeStruct((_NC, NP, D), F32),
        mesh=_mesh(),
        scratch_types=[
            pltpu.VMEM_SHARED((NP, D), F32),
            pltpu.VMEM((_CK,), jnp.int32),
            pltpu.VMEM((_CK,), jnp.int32),
            pltpu.VMEM((_CK, D), F32),
            pltpu.VMEM((ZR, D), F32),
            pltpu.SemaphoreType.DMA,
        ],
    )
    def k(t_hbm, src_hbm, dst_hbm, out_hbm, acc, src_v, dst_v, rows_v, zero_v, sem):
        c = lax.axis_index("c")
        s = lax.axis_index("s")

        @pl.loop(0, ZR)
        def _(i):
            @pl.loop(0, D // 16)
            def _(j):
                zero_v[i, pl.ds(j * 16, 16)] = jnp.zeros((16,), F32)

        for kk in range(RP // ZR):
            pltpu.sync_copy(zero_v, acc.at[pl.ds(s * RP + kk * ZR, ZR)])
        plsc.subcore_barrier()

        base0 = c * E2 + s * EP

        @pl.loop(0, NCH)
        def _(j):
            b = base0 + j * _CK
            pltpu.sync_copy(src_hbm.at[pl.ds(b, _CK)], src_v)
            pltpu.sync_copy(dst_hbm.at[pl.ds(b, _CK)], dst_v)
            pltpu.async_copy(t_hbm.at[src_v], rows_v, sem).wait()
            pltpu.sync_copy(rows_v, acc.at[dst_v], add=True)

        plsc.subcore_barrier()
        for kk in range(RP // ZR):
            r0 = s * RP + kk * ZR
            pltpu.sync_copy(acc.at[pl.ds(r0, ZR)], out_hbm.at[c, pl.ds(r0, ZR)])

    return k


def _tc_dis(degp, N, D):
    def body(degp_ref, o_ref):
        d = degp_ref[0, :, 0:1] + degp_ref[1, :, 0:1] + 1.0
        o_ref[...] = jnp.broadcast_to(lax.rsqrt(d)[:N], (N, D))

    return pl.pallas_call(
        body, out_shape=jax.ShapeDtypeStruct((N, D), F32))(degp)


def _tc_layer0(x, dis_b, W):
    def body(x_ref, dis_ref, W_ref, o_ref):
        o_ref[...] = jnp.dot(x_ref[...], W_ref[...],
                             preferred_element_type=F32) * dis_ref[...]

    return pl.pallas_call(
        body, out_shape=jax.ShapeDtypeStruct(x.shape, F32))(x, dis_b, W)


def _tc_layer(S, t, dis_b, b, W):
    N = t.shape[0]

    def body(S_ref, t_ref, dis_ref, b_ref, W_ref, o_ref):
        agg = (S_ref[0, :N] + S_ref[1, :N] + t_ref[...]) * dis_ref[...]
        h = jnp.maximum(agg + b_ref[...], 0.0)
        o_ref[...] = jnp.dot(h, W_ref[...],
                             preferred_element_type=F32) * dis_ref[...]

    return pl.pallas_call(
        body, out_shape=jax.ShapeDtypeStruct(t.shape, F32))(S, t, dis_b, b, W)


def _tc_head(S, t, dis_b, b, head_W, head_b):
    N = t.shape[0]

    def body(S_ref, t_ref, dis_ref, b_ref, hW_ref, hb_ref, o_ref):
        agg = (S_ref[0, :N] + S_ref[1, :N] + t_ref[...]) * dis_ref[...]
        h = jnp.maximum(agg + b_ref[...], 0.0)
        o_ref[...] = jnp.maximum(
            jnp.dot(h, hW_ref[...], preferred_element_type=F32) + hb_ref[...],
            0.0)

    return pl.pallas_call(
        body, out_shape=jax.ShapeDtypeStruct((N, 1), F32))(
            S, t, dis_b, b, head_W, head_b)


def kernel(x, edge_index, Ws, bs, head_W, head_b):
    N, D = x.shape
    E = edge_index.shape[1]
    L = Ws.shape[0]
    src = edge_index[0]
    dst = edge_index[1]

    NP = -(-N // (16 * _NS)) * (16 * _NS)

    degp = _sc_degree(E, NP)(dst)
    dis_b = _tc_dis(degp, N, D)

    t = _tc_layer0(x, dis_b, Ws[0])
    scat = _sc_scatter(NP, D, E)
    out = None
    for i in range(L):
        S = scat(t, src, dst)
        b = bs[i].reshape(1, D)
        if i + 1 < L:
            t = _tc_layer(S, t, dis_b, b, Ws[i + 1])
        else:
            out = _tc_head(S, t, dis_b, b, head_W, head_b.reshape(1, 1))
    return out


# speedup vs baseline: 18.6830x; 2.3660x over previous
import functools

import jax
import jax.numpy as jnp
from jax import lax
from jax.experimental import pallas as pl
from jax.experimental.pallas import tpu as pltpu
from jax.experimental.pallas import tpu_sc as plsc

F32 = jnp.float32
_NC = 2
_NS = 16
_CK = 80


def _mesh():
    return plsc.VectorSubcoreMesh(core_axis_name="c", subcore_axis_name="s")


@functools.cache
def _sc_scatter(NP, D, E):
    E2 = E // _NC
    EP = E2 // _NS
    NCH = EP // _CK
    RP = NP // _NS

    @functools.partial(
        pl.kernel,
        out_type=jax.ShapeDtypeStruct((_NC, NP, D), F32),
        mesh=_mesh(),
        scratch_types=[
            pltpu.VMEM_SHARED((NP, D), F32),
            pltpu.VMEM((NCH, _CK), jnp.int32),
            pltpu.VMEM((_CK,), jnp.int32),
            pltpu.VMEM((_CK,), jnp.int32),
            pltpu.VMEM((_CK, D), F32),
            pltpu.VMEM((_CK, D), F32),
            pltpu.SemaphoreType.DMA,
            pltpu.SemaphoreType.DMA,
            pltpu.SemaphoreType.DMA,
            pltpu.SemaphoreType.DMA,
        ],
    )
    def k(t_hbm, src_hbm, dst_hbm, out_hbm, acc,
          src_v, dstb0, dstb1, rows0, rows1, gs0, gs1, ds0, ds1):
        c = lax.axis_index("c")
        s = lax.axis_index("s")
        pltpu.sync_copy(src_hbm.at[c, s], src_v)
        base0 = c * E2 + s * EP

        def dst_start(j, buf, sem):
            pltpu.async_copy(dst_hbm.at[pl.ds(base0 + j * _CK, _CK)], buf, sem)

        def dst_wait(j, buf, sem):
            pltpu.make_async_copy(
                dst_hbm.at[pl.ds(base0 + j * _CK, _CK)], buf, sem).wait()

        @pl.loop(0, _CK)
        def _(i):
            @pl.loop(0, D // 16)
            def _(j):
                rows0[i, pl.ds(j * 16, 16)] = jnp.zeros((16,), F32)

        for kk in range(RP // _CK):
            pltpu.sync_copy(rows0, acc.at[pl.ds(s * RP + kk * _CK, _CK)])

        dst_start(0, dstb0, ds0)
        dst_start(1, dstb1, ds1)
        pltpu.async_copy(t_hbm.at[src_v.at[0]], rows0, gs0)
        pltpu.async_copy(t_hbm.at[src_v.at[1]], rows1, gs1)
        plsc.subcore_barrier()

        @pl.loop(0, NCH - 1, step=2)
        def _(j):
            pltpu.make_async_copy(t_hbm.at[src_v.at[j]], rows0, gs0).wait()
            dst_wait(j, dstb0, ds0)
            pltpu.sync_copy(rows0, acc.at[dstb0], add=True)

            @pl.when(j + 2 < NCH)
            def _():
                dst_start(j + 2, dstb0, ds0)
                pltpu.async_copy(t_hbm.at[src_v.at[j + 2]], rows0, gs0)

            pltpu.make_async_copy(t_hbm.at[src_v.at[j + 1]], rows1, gs1).wait()
            dst_wait(j + 1, dstb1, ds1)
            pltpu.sync_copy(rows1, acc.at[dstb1], add=True)

            @pl.when(j + 3 < NCH)
            def _():
                dst_start(j + 3, dstb1, ds1)
                pltpu.async_copy(t_hbm.at[src_v.at[j + 3]], rows1, gs1)

        if NCH % 2 == 1:
            pltpu.make_async_copy(t_hbm.at[src_v.at[NCH - 1]], rows0, gs0).wait()
            dst_wait(NCH - 1, dstb0, ds0)
            pltpu.sync_copy(rows0, acc.at[dstb0], add=True)

        plsc.subcore_barrier()
        for kk in range(RP // _CK):
            r0 = s * RP + kk * _CK
            buf = rows0 if kk % 2 == 0 else rows1
            pltpu.sync_copy(acc.at[pl.ds(r0, _CK)], buf)
            pltpu.sync_copy(buf, out_hbm.at[c, pl.ds(r0, _CK)])

    return k


def _tc_dis(degp, N, D):
    def body(degp_ref, o_ref):
        d = degp_ref[0, :, 0:1] + degp_ref[1, :, 0:1] + 1.0
        o_ref[...] = jnp.broadcast_to(lax.rsqrt(d)[:N], (N, D))

    return pl.pallas_call(
        body, out_shape=jax.ShapeDtypeStruct((N, D), F32))(degp)


def _tc_layer0(x, dis_b, W):
    def body(x_ref, dis_ref, W_ref, o_ref):
        o_ref[...] = jnp.dot(x_ref[...], W_ref[...],
                             preferred_element_type=F32) * dis_ref[...]

    return pl.pallas_call(
        body, out_shape=jax.ShapeDtypeStruct(x.shape, F32))(x, dis_b, W)


def _tc_layer(S, t, dis_b, b, W):
    N = t.shape[0]

    def body(S_ref, t_ref, dis_ref, b_ref, W_ref, o_ref):
        agg = (S_ref[0, :N] + S_ref[1, :N] + t_ref[...]) * dis_ref[...]
        h = jnp.maximum(agg + b_ref[...], 0.0)
        o_ref[...] = jnp.dot(h, W_ref[...],
                             preferred_element_type=F32) * dis_ref[...]

    return pl.pallas_call(
        body, out_shape=jax.ShapeDtypeStruct(t.shape, F32))(S, t, dis_b, b, W)


def _tc_head(S, t, dis_b, b, head_W, head_b):
    N = t.shape[0]

    def body(S_ref, t_ref, dis_ref, b_ref, hW_ref, hb_ref, o_ref):
        agg = (S_ref[0, :N] + S_ref[1, :N] + t_ref[...]) * dis_ref[...]
        h = jnp.maximum(agg + b_ref[...], 0.0)
        o_ref[...] = jnp.maximum(
            jnp.dot(h, hW_ref[...], preferred_element_type=F32) + hb_ref[...],
            0.0)

    return pl.pallas_call(
        body, out_shape=jax.ShapeDtypeStruct((N, 1), F32))(
            S, t, dis_b, b, head_W, head_b)


def kernel(x, edge_index, Ws, bs, head_W, head_b):
    N, D = x.shape
    E = edge_index.shape[1]
    L = Ws.shape[0]
    NCH = (E // (_NC * _NS)) // _CK
    src = edge_index[0].reshape(_NC, _NS, NCH, _CK)
    dst = edge_index[1]

    NP = -(-N // (16 * _NS)) * (16 * _NS)

    scat = _sc_scatter(NP, D, E)
    degp = scat(jnp.ones((N, D), F32), src, dst)
    dis_b = _tc_dis(degp, N, D)

    t = _tc_layer0(x, dis_b, Ws[0])
    out = None
    for i in range(L):
        S = scat(t, src, dst)
        b = bs[i].reshape(1, D)
        if i + 1 < L:
            t = _tc_layer(S, t, dis_b, b, Ws[i + 1])
        else:
            out = _tc_head(S, t, dis_b, b, head_W, head_b.reshape(1, 1))
    return out


# speedup vs baseline: 22.9997x; 1.2311x over previous
import functools

import jax
import jax.numpy as jnp
from jax import lax
from jax.experimental import pallas as pl
from jax.experimental.pallas import tpu as pltpu
from jax.experimental.pallas import tpu_sc as plsc

F32 = jnp.float32
_NC = 2
_NS = 16
_CK = 80


def _mesh():
    return plsc.VectorSubcoreMesh(core_axis_name="c", subcore_axis_name="s")


@functools.cache
def _sc_scatter(NP, D, E):
    E2 = E // _NC
    EP = E2 // _NS
    NCH = EP // _CK
    RP = NP // _NS

    DEP = 3

    @functools.partial(
        pl.kernel,
        out_type=jax.ShapeDtypeStruct((_NC, NP, D), F32),
        mesh=_mesh(),
        scratch_types=[
            pltpu.VMEM_SHARED((NP, D), F32),
            pltpu.VMEM((NCH, _CK), jnp.int32),
            [pltpu.VMEM((_CK,), jnp.int32) for _ in range(DEP)],
            [pltpu.VMEM((_CK, D), F32) for _ in range(DEP)],
            [pltpu.SemaphoreType.DMA for _ in range(DEP)],
            [pltpu.SemaphoreType.DMA for _ in range(DEP)],
            [pltpu.SemaphoreType.DMA for _ in range(DEP)],
        ],
    )
    def k(t_hbm, src_hbm, dst_hbm, out_hbm, acc, src_v, dstb, rows, gs, ds, ss):
        c = lax.axis_index("c")
        s = lax.axis_index("s")
        pltpu.sync_copy(src_hbm.at[c, s], src_v)
        base0 = c * E2 + s * EP

        def dst_start(j, b):
            pltpu.async_copy(dst_hbm.at[pl.ds(base0 + j * _CK, _CK)],
                             dstb[b], ds[b])

        def dst_wait(j, b):
            pltpu.make_async_copy(dst_hbm.at[pl.ds(base0 + j * _CK, _CK)],
                                  dstb[b], ds[b]).wait()

        def gat_start(j, b):
            pltpu.async_copy(t_hbm.at[src_v.at[j]], rows[b], gs[b])

        def gat_wait(j, b):
            pltpu.make_async_copy(t_hbm.at[src_v.at[j]], rows[b], gs[b]).wait()

        def scat_start(b):
            pltpu.async_copy(rows[b], acc.at[dstb[b]], ss[b], add=True)

        def scat_wait(b):
            pltpu.make_async_copy(rows[b], acc.at[dstb[b]], ss[b]).wait()

        @pl.loop(0, _CK)
        def _(i):
            @pl.loop(0, D // 16)
            def _(j):
                rows[0][i, pl.ds(j * 16, 16)] = jnp.zeros((16,), F32)

        for kk in range(RP // _CK):
            pltpu.sync_copy(rows[0], acc.at[pl.ds(s * RP + kk * _CK, _CK)])

        dst_start(0, 0)
        gat_start(0, 0)
        dst_start(1, 1)
        gat_start(1, 1)
        plsc.subcore_barrier()

        def chunk_step(jj, b, bpre, prefetch, may_be_first):
            if prefetch:
                @pl.when(jj + 2 < NCH)
                def _():
                    if may_be_first:
                        @pl.when(jj >= 1)
                        def _():
                            scat_wait(bpre)
                    else:
                        scat_wait(bpre)
                    dst_start(jj + 2, bpre)
                    gat_start(jj + 2, bpre)
            gat_wait(jj, b)
            dst_wait(jj, b)
            scat_start(b)

        @pl.loop(0, NCH - 2, step=DEP)
        def _(j):
            for b in range(DEP):
                chunk_step(j + b, b, (b + 2) % DEP, True, b == 0)

        for jj in range(NCH - 2, NCH):
            chunk_step(jj, jj % DEP, 0, False, False)
        for b in range(DEP):
            scat_wait(b)

        plsc.subcore_barrier()
        nwb = RP // _CK
        for kk in range(nwb):
            b = kk % DEP
            r0 = s * RP + kk * _CK
            if kk >= DEP:
                rp = s * RP + (kk - DEP) * _CK
                pltpu.make_async_copy(rows[b], out_hbm.at[c, pl.ds(rp, _CK)],
                                      gs[b]).wait()
            pltpu.sync_copy(acc.at[pl.ds(r0, _CK)], rows[b])
            pltpu.async_copy(rows[b], out_hbm.at[c, pl.ds(r0, _CK)], gs[b])
        for kk in range(max(nwb - DEP, 0), nwb):
            b = kk % DEP
            r0 = s * RP + kk * _CK
            pltpu.make_async_copy(rows[b], out_hbm.at[c, pl.ds(r0, _CK)],
                                  gs[b]).wait()

    return k


def _tc_dis(degp, N, D):
    def body(degp_ref, o_ref):
        d = degp_ref[0, :, 0:1] + degp_ref[1, :, 0:1] + 1.0
        o_ref[...] = jnp.broadcast_to(lax.rsqrt(d)[:N], (N, D))

    return pl.pallas_call(
        body, out_shape=jax.ShapeDtypeStruct((N, D), F32))(degp)


def _tc_layer0(x, dis_b, W):
    def body(x_ref, dis_ref, W_ref, o_ref):
        o_ref[...] = jnp.dot(x_ref[...], W_ref[...],
                             preferred_element_type=F32) * dis_ref[...]

    return pl.pallas_call(
        body, out_shape=jax.ShapeDtypeStruct(x.shape, F32))(x, dis_b, W)


def _tc_layer(S, t, dis_b, b, W):
    N = t.shape[0]

    def body(S_ref, t_ref, dis_ref, b_ref, W_ref, o_ref):
        agg = (S_ref[0, :N] + S_ref[1, :N] + t_ref[...]) * dis_ref[...]
        h = jnp.maximum(agg + b_ref[...], 0.0)
        o_ref[...] = jnp.dot(h, W_ref[...],
                             preferred_element_type=F32) * dis_ref[...]

    return pl.pallas_call(
        body, out_shape=jax.ShapeDtypeStruct(t.shape, F32))(S, t, dis_b, b, W)


def _tc_head(S, t, dis_b, b, head_W, head_b):
    N = t.shape[0]

    def body(S_ref, t_ref, dis_ref, b_ref, hW_ref, hb_ref, o_ref):
        agg = (S_ref[0, :N] + S_ref[1, :N] + t_ref[...]) * dis_ref[...]
        h = jnp.maximum(agg + b_ref[...], 0.0)
        o_ref[...] = jnp.maximum(
            jnp.dot(h, hW_ref[...], preferred_element_type=F32) + hb_ref[...],
            0.0)

    return pl.pallas_call(
        body, out_shape=jax.ShapeDtypeStruct((N, 1), F32))(
            S, t, dis_b, b, head_W, head_b)


def kernel(x, edge_index, Ws, bs, head_W, head_b):
    N, D = x.shape
    E = edge_index.shape[1]
    L = Ws.shape[0]
    NCH = (E // (_NC * _NS)) // _CK
    src = edge_index[0].reshape(_NC, _NS, NCH, _CK)
    dst = edge_index[1]

    NP = -(-N // (16 * _NS)) * (16 * _NS)

    scat = _sc_scatter(NP, D, E)
    degp = scat(jnp.ones((N, D), F32), src, dst)
    dis_b = _tc_dis(degp, N, D)

    t = _tc_layer0(x, dis_b, Ws[0])
    out = None
    for i in range(L):
        S = scat(t, src, dst)
        b = bs[i].reshape(1, D)
        if i + 1 < L:
            t = _tc_layer(S, t, dis_b, b, Ws[i + 1])
        else:
            out = _tc_head(S, t, dis_b, b, head_W, head_b.reshape(1, 1))
    return out


# speedup vs baseline: 23.2638x; 1.0115x over previous
import functools

import jax
import jax.numpy as jnp
from jax import lax
from jax.experimental import pallas as pl
from jax.experimental.pallas import tpu as pltpu
from jax.experimental.pallas import tpu_sc as plsc

F32 = jnp.float32
_NC = 2
_NS = 16
_CK = 80


def _mesh():
    return plsc.VectorSubcoreMesh(core_axis_name="c", subcore_axis_name="s")


@functools.cache
def _sc_scatter(NP, D, E):
    E2 = E // _NC
    EP = E2 // _NS
    NCH = EP // _CK
    RP = NP // _NS

    DEP = 3

    @functools.partial(
        pl.kernel,
        out_type=jax.ShapeDtypeStruct((_NC, NP, D), F32),
        mesh=_mesh(),
        scratch_types=[
            pltpu.VMEM_SHARED((NP, D), F32),
            pltpu.VMEM((NCH, _CK), jnp.int32),
            [pltpu.VMEM((_CK,), jnp.int32) for _ in range(DEP)],
            [pltpu.VMEM((_CK, D), F32) for _ in range(DEP)],
            [pltpu.SemaphoreType.DMA for _ in range(DEP)],
            [pltpu.SemaphoreType.DMA for _ in range(DEP)],
            [pltpu.SemaphoreType.DMA for _ in range(DEP)],
        ],
    )
    def k(t_hbm, src_hbm, dst_hbm, out_hbm, acc, src_v, dstb, rows, gs, ds, ss):
        c = lax.axis_index("c")
        s = lax.axis_index("s")
        src_cp = pltpu.async_copy(src_hbm.at[c, s], src_v, ss[0])
        base0 = c * E2 + s * EP

        def dst_start(j, b):
            pltpu.async_copy(dst_hbm.at[pl.ds(base0 + j * _CK, _CK)],
                             dstb[b], ds[b])

        def dst_wait(j, b):
            pltpu.make_async_copy(dst_hbm.at[pl.ds(base0 + j * _CK, _CK)],
                                  dstb[b], ds[b]).wait()

        def gat_start(j, b):
            pltpu.async_copy(t_hbm.at[src_v.at[j]], rows[b], gs[b])

        def gat_wait(j, b):
            pltpu.make_async_copy(t_hbm.at[src_v.at[j]], rows[b], gs[b]).wait()

        def scat_start(b):
            pltpu.async_copy(rows[b], acc.at[dstb[b]], ss[b], add=True)

        def scat_wait(b):
            pltpu.make_async_copy(rows[b], acc.at[dstb[b]], ss[b]).wait()

        @pl.loop(0, _CK)
        def _(i):
            @pl.loop(0, D // 16)
            def _(j):
                rows[0][i, pl.ds(j * 16, 16)] = jnp.zeros((16,), F32)

        for kk in range(RP // _CK):
            pltpu.sync_copy(rows[0], acc.at[pl.ds(s * RP + kk * _CK, _CK)])

        src_cp.wait()
        dst_start(0, 0)
        gat_start(0, 0)
        dst_start(1, 1)
        gat_start(1, 1)
        plsc.subcore_barrier()

        def chunk_step(jj, b, bpre, prefetch, may_be_first):
            if prefetch:
                @pl.when(jj + 2 < NCH)
                def _():
                    if may_be_first:
                        @pl.when(jj >= 1)
                        def _():
                            scat_wait(bpre)
                    else:
                        scat_wait(bpre)
                    dst_start(jj + 2, bpre)
                    gat_start(jj + 2, bpre)
            gat_wait(jj, b)
            dst_wait(jj, b)
            scat_start(b)

        @pl.loop(0, NCH - 2, step=DEP)
        def _(j):
            for b in range(DEP):
                chunk_step(j + b, b, (b + 2) % DEP, True, b == 0)

        for jj in range(NCH - 2, NCH):
            chunk_step(jj, jj % DEP, 0, False, False)
        for b in range(DEP):
            scat_wait(b)

        plsc.subcore_barrier()
        nwb = RP // _CK
        for kk in range(nwb):
            b = kk % DEP
            r0 = s * RP + kk * _CK
            if kk >= DEP:
                rp = s * RP + (kk - DEP) * _CK
                pltpu.make_async_copy(rows[b], out_hbm.at[c, pl.ds(rp, _CK)],
                                      gs[b]).wait()
            pltpu.sync_copy(acc.at[pl.ds(r0, _CK)], rows[b])
            pltpu.async_copy(rows[b], out_hbm.at[c, pl.ds(r0, _CK)], gs[b])
        for kk in range(max(nwb - DEP, 0), nwb):
            b = kk % DEP
            r0 = s * RP + kk * _CK
            pltpu.make_async_copy(rows[b], out_hbm.at[c, pl.ds(r0, _CK)],
                                  gs[b]).wait()

    return k


def _tc_dis(degp, N, D):
    def body(degp_ref, o_ref):
        d = degp_ref[0, :, 0:1] + degp_ref[1, :, 0:1] + 1.0
        o_ref[...] = lax.rsqrt(d)[:N]

    return pl.pallas_call(
        body, out_shape=jax.ShapeDtypeStruct((N, 1), F32))(degp)


def _tc_layer0(x, dis_b, W):
    def body(x_ref, dis_ref, W_ref, o_ref):
        o_ref[...] = jnp.dot(x_ref[...], W_ref[...],
                             preferred_element_type=F32) * dis_ref[...]

    return pl.pallas_call(
        body, out_shape=jax.ShapeDtypeStruct(x.shape, F32))(x, dis_b, W)


def _tc_layer(S, t, dis_b, b, W):
    N = t.shape[0]

    def body(S_ref, t_ref, dis_ref, b_ref, W_ref, o_ref):
        agg = (S_ref[0, :N] + S_ref[1, :N] + t_ref[...]) * dis_ref[...]
        h = jnp.maximum(agg + b_ref[...], 0.0)
        o_ref[...] = jnp.dot(h, W_ref[...],
                             preferred_element_type=F32) * dis_ref[...]

    return pl.pallas_call(
        body, out_shape=jax.ShapeDtypeStruct(t.shape, F32))(S, t, dis_b, b, W)


def _tc_head(S, t, dis_b, b, head_W, head_b):
    N = t.shape[0]

    def body(S_ref, t_ref, dis_ref, b_ref, hW_ref, hb_ref, o_ref):
        agg = (S_ref[0, :N] + S_ref[1, :N] + t_ref[...]) * dis_ref[...]
        h = jnp.maximum(agg + b_ref[...], 0.0)
        o_ref[...] = jnp.maximum(
            jnp.dot(h, hW_ref[...], preferred_element_type=F32) + hb_ref[...],
            0.0)

    return pl.pallas_call(
        body, out_shape=jax.ShapeDtypeStruct((N, 1), F32))(
            S, t, dis_b, b, head_W, head_b)


def kernel(x, edge_index, Ws, bs, head_W, head_b):
    N, D = x.shape
    E = edge_index.shape[1]
    L = Ws.shape[0]
    NCH = (E // (_NC * _NS)) // _CK
    src = edge_index[0].reshape(_NC, _NS, NCH, _CK)
    dst = edge_index[1]

    NP = -(-N // (16 * _NS)) * (16 * _NS)

    scat = _sc_scatter(NP, D, E)
    degp = scat(jnp.ones((N, D), F32), src, dst)
    dis_b = _tc_dis(degp, N, D)

    t = _tc_layer0(x, dis_b, Ws[0])
    out = None
    for i in range(L):
        S = scat(t, src, dst)
        b = bs[i].reshape(1, D)
        if i + 1 < L:
            t = _tc_layer(S, t, dis_b, b, Ws[i + 1])
        else:
            out = _tc_head(S, t, dis_b, b, head_W, head_b.reshape(1, 1))
    return out


# speedup vs baseline: 23.7051x; 1.0190x over previous
import functools

import jax
import jax.numpy as jnp
from jax import lax
from jax.experimental import pallas as pl
from jax.experimental.pallas import tpu as pltpu
from jax.experimental.pallas import tpu_sc as plsc

F32 = jnp.float32
_NC = 2
_NS = 16
_CK = 80


def _mesh():
    return plsc.VectorSubcoreMesh(core_axis_name="c", subcore_axis_name="s")


@functools.cache
def _sc_scatter(NP, D, E):
    E2 = E // _NC
    EP = E2 // _NS
    NCH = EP // _CK
    RP = NP // _NS

    DEP = 3

    @functools.partial(
        pl.kernel,
        out_type=jax.ShapeDtypeStruct((_NC, NP, D), F32),
        mesh=_mesh(),
        scratch_types=[
            pltpu.VMEM_SHARED((NP, D), F32),
            pltpu.VMEM((NCH, _CK), jnp.int32),
            [pltpu.VMEM((_CK,), jnp.int32) for _ in range(DEP)],
            [pltpu.VMEM((_CK, D), F32) for _ in range(DEP)],
            [pltpu.SemaphoreType.DMA for _ in range(DEP)],
            [pltpu.SemaphoreType.DMA for _ in range(DEP)],
            [pltpu.SemaphoreType.DMA for _ in range(DEP)],
        ],
    )
    def k(t_hbm, src_hbm, dst_hbm, out_hbm, acc, src_v, dstb, rows, gs, ds, ss):
        c = lax.axis_index("c")
        s = lax.axis_index("s")
        src_cp = pltpu.async_copy(src_hbm.at[c, s], src_v, ss[0])
        base0 = c * E2 + s * EP

        def dst_start(j, b):
            pltpu.async_copy(dst_hbm.at[pl.ds(base0 + j * _CK, _CK)],
                             dstb[b], ds[b])

        def dst_wait(j, b):
            pltpu.make_async_copy(dst_hbm.at[pl.ds(base0 + j * _CK, _CK)],
                                  dstb[b], ds[b]).wait()

        def gat_start(j, b):
            pltpu.async_copy(t_hbm.at[src_v.at[j]], rows[b], gs[b])

        def gat_wait(j, b):
            pltpu.make_async_copy(t_hbm.at[src_v.at[j]], rows[b], gs[b]).wait()

        def scat_start(b):
            pltpu.async_copy(rows[b], acc.at[dstb[b]], ss[b], add=True)

        def scat_wait(b):
            pltpu.make_async_copy(rows[b], acc.at[dstb[b]], ss[b]).wait()

        @pl.loop(0, _CK)
        def _(i):
            @pl.loop(0, D // 16)
            def _(j):
                rows[2][i, pl.ds(j * 16, 16)] = jnp.zeros((16,), F32)

        for kk in range(RP // _CK):
            pltpu.async_copy(rows[2], acc.at[pl.ds(s * RP + kk * _CK, _CK)],
                             ss[1])

        src_cp.wait()
        dst_start(0, 0)
        gat_start(0, 0)
        dst_start(1, 1)
        gat_start(1, 1)
        for kk in range(RP // _CK):
            pltpu.make_async_copy(rows[2],
                                  acc.at[pl.ds(s * RP + kk * _CK, _CK)],
                                  ss[1]).wait()
        plsc.subcore_barrier()

        def chunk_step(jj, b, bpre, prefetch, may_be_first):
            if prefetch:
                @pl.when(jj + 2 < NCH)
                def _():
                    if may_be_first:
                        @pl.when(jj >= 1)
                        def _():
                            scat_wait(bpre)
                    else:
                        scat_wait(bpre)
                    dst_start(jj + 2, bpre)
                    gat_start(jj + 2, bpre)
            gat_wait(jj, b)
            dst_wait(jj, b)
            scat_start(b)

        @pl.loop(0, NCH - 2, step=DEP)
        def _(j):
            for b in range(DEP):
                chunk_step(j + b, b, (b + 2) % DEP, True, b == 0)

        for jj in range(NCH - 2, NCH):
            chunk_step(jj, jj % DEP, 0, False, False)
        for b in range(DEP):
            scat_wait(b)

        plsc.subcore_barrier()
        nwb = RP // _CK
        for kk in range(nwb):
            b = kk % DEP
            r0 = s * RP + kk * _CK
            if kk >= DEP:
                rp = s * RP + (kk - DEP) * _CK
                pltpu.make_async_copy(rows[b], out_hbm.at[c, pl.ds(rp, _CK)],
                                      gs[b]).wait()
            pltpu.sync_copy(acc.at[pl.ds(r0, _CK)], rows[b])
            pltpu.async_copy(rows[b], out_hbm.at[c, pl.ds(r0, _CK)], gs[b])
        for kk in range(max(nwb - DEP, 0), nwb):
            b = kk % DEP
            r0 = s * RP + kk * _CK
            pltpu.make_async_copy(rows[b], out_hbm.at[c, pl.ds(r0, _CK)],
                                  gs[b]).wait()

    return k


def _tc_dis(degp, N, D):
    def body(degp_ref, o_ref):
        d = degp_ref[0, :, 0:1] + degp_ref[1, :, 0:1] + 1.0
        o_ref[...] = lax.rsqrt(d)[:N]

    return pl.pallas_call(
        body, out_shape=jax.ShapeDtypeStruct((N, 1), F32))(degp)


def _tc_layer0(x, dis_b, W):
    def body(x_ref, dis_ref, W_ref, o_ref):
        o_ref[...] = jnp.dot(x_ref[...], W_ref[...],
                             preferred_element_type=F32) * dis_ref[...]

    return pl.pallas_call(
        body, out_shape=jax.ShapeDtypeStruct(x.shape, F32))(x, dis_b, W)


def _tc_layer(S, t, dis_b, b, W):
    N = t.shape[0]

    def body(S_ref, t_ref, dis_ref, b_ref, W_ref, o_ref):
        agg = (S_ref[0, :N] + S_ref[1, :N] + t_ref[...]) * dis_ref[...]
        h = jnp.maximum(agg + b_ref[...], 0.0)
        o_ref[...] = jnp.dot(h, W_ref[...],
                             preferred_element_type=F32) * dis_ref[...]

    return pl.pallas_call(
        body, out_shape=jax.ShapeDtypeStruct(t.shape, F32))(S, t, dis_b, b, W)


def _tc_head(S, t, dis_b, b, head_W, head_b):
    N = t.shape[0]

    def body(S_ref, t_ref, dis_ref, b_ref, hW_ref, hb_ref, o_ref):
        agg = (S_ref[0, :N] + S_ref[1, :N] + t_ref[...]) * dis_ref[...]
        h = jnp.maximum(agg + b_ref[...], 0.0)
        o_ref[...] = jnp.maximum(
            jnp.dot(h, hW_ref[...], preferred_element_type=F32) + hb_ref[...],
            0.0)

    return pl.pallas_call(
        body, out_shape=jax.ShapeDtypeStruct((N, 1), F32))(
            S, t, dis_b, b, head_W, head_b)


def kernel(x, edge_index, Ws, bs, head_W, head_b):
    N, D = x.shape
    E = edge_index.shape[1]
    L = Ws.shape[0]
    NCH = (E // (_NC * _NS)) // _CK
    src = edge_index[0].reshape(_NC, _NS, NCH, _CK)
    dst = edge_index[1]

    NP = -(-N // (16 * _NS)) * (16 * _NS)

    scat = _sc_scatter(NP, D, E)
    degp = scat(jnp.ones((N, D), F32), src, dst)
    dis_b = _tc_dis(degp, N, D)

    t = _tc_layer0(x, dis_b, Ws[0])
    out = None
    for i in range(L):
        S = scat(t, src, dst)
        b = bs[i].reshape(1, D)
        if i + 1 < L:
            t = _tc_layer(S, t, dis_b, b, Ws[i + 1])
        else:
            out = _tc_head(S, t, dis_b, b, head_W, head_b.reshape(1, 1))
    return out
